# Initial kernel scaffold; baseline (speedup 1.0000x reference)
import jax
import jax.numpy as jnp
from jax.experimental import pallas as pl


def kernel(x, table, W, b):
    raise NotImplementedError("write your pallas kernel here")


# speedup vs baseline: 2.4785x; 2.4785x over previous
import jax
import jax.numpy as jnp
from jax import lax
from jax.experimental import pallas as pl
from jax.experimental.pallas import tpu as pltpu
from jax.experimental.pallas import tpu_sc as plsc

_V = 1_000_000
_E = 64
_B = 16384
_H = 200
_PW = 16

_NC = 2
_NS = 16
_NW = _NC * _NS
_RPT = _B // _NW
_GR = 16
_NG = _RPT // _GR
_CPG = (_GR * _H) // 128
_CROWS_PT = (_B * _H) // 128 // _NW


def _proj_body(t_ref, w_ref, o_ref):
    o_ref[...] = jnp.dot(t_ref[...], w_ref[...],
                         preferred_element_type=jnp.float32)


def _project(table, wp):
    blk = 8000
    return pl.pallas_call(
        _proj_body,
        grid=(_V // blk,),
        in_specs=[
            pl.BlockSpec((blk, _E), lambda i: (i, 0)),
            pl.BlockSpec((_E, _PW), lambda i: (0, 0)),
        ],
        out_specs=pl.BlockSpec((blk, _PW), lambda i: (i, 0)),
        out_shape=jax.ShapeDtypeStruct((_V, _PW), jnp.float32),
    )(table, wp)


_GPS = 8
_SB = _NG // _GPS
_SBROWS = _GPS * _CPG


def _bag_body(xr_hbm, proj_hbm, bpad_hbm, out_hbm,
              idx_v, rows_v, out_v, bias_v, gsem):
    wid = lax.axis_index("s") * _NC + lax.axis_index("c")
    pltpu.sync_copy(bpad_hbm, bias_v)
    cbase = wid * _CROWS_PT

    def superblock(sb, carry):
        pltpu.sync_copy(xr_hbm.at[pl.ds(cbase + sb * _SBROWS, _SBROWS)],
                        idx_v)

        def group(gg, carry2):
            cps = []
            for j in range(_CPG):
                cps.append(pltpu.async_copy(
                    proj_hbm.at[idx_v.at[gg * _CPG + j]],
                    rows_v.at[pl.ds(j * 128, 128)], gsem))
            for cp in cps:
                cp.wait()
            for r in range(_GR):
                def red(l8, a):
                    base = r * _H + l8 * 8
                    for u in range(8):
                        a = a + rows_v[base + u]
                    return a
                acc = lax.fori_loop(0, _H // 8, red, bias_v[...])
                out_v[sb * (_GPS * _GR) + gg * _GR + r] = acc
            return carry2

        lax.fori_loop(0, _GPS, group, 0)
        return carry

    lax.fori_loop(0, _SB, superblock, 0)
    pltpu.sync_copy(out_v, out_hbm.at[pl.ds(wid * _RPT, _RPT)])


def _bag(xr, proj, bpad):
    mesh = plsc.VectorSubcoreMesh(core_axis_name="c", subcore_axis_name="s")
    f = pl.kernel(
        _bag_body,
        mesh=mesh,
        out_type=jax.ShapeDtypeStruct((_B, _PW), jnp.float32),
        scratch_types=[
            pltpu.VMEM((_SBROWS, 128), jnp.int32),
            pltpu.VMEM((_GR * _H, _PW), jnp.float32),
            pltpu.VMEM((_RPT, _PW), jnp.float32),
            pltpu.VMEM((16,), jnp.float32),
            pltpu.SemaphoreType.DMA,
        ],
        compiler_params=pltpu.CompilerParams(use_tc_tiling_on_sc=False),
    )
    return f(xr, proj, bpad)


def kernel(x, table, W, b):
    c = W.shape[0]
    xr = x.astype(jnp.int32).reshape(-1, 128)
    wp = jnp.pad(W.T.astype(jnp.float32) * (1.0 / _H),
                 ((0, 0), (0, _PW - c)))
    bp = jnp.pad(b.astype(jnp.float32), (0, _PW - c))
    proj = _project(table, wp)
    out = _bag(xr, proj, bp)
    return out[:, :c]


# speedup vs baseline: 3.3808x; 1.3640x over previous
import jax
import jax.numpy as jnp
from jax import lax
from jax.experimental import pallas as pl
from jax.experimental.pallas import tpu as pltpu
from jax.experimental.pallas import tpu_sc as plsc

_V = 1_000_000
_E = 64
_B = 16384
_H = 200
_PW = 16

_NC = 2
_NS = 16
_NW = _NC * _NS
_RPT = _B // _NW
_GR = 16
_NG = _RPT // _GR
_CPG = (_GR * _H) // 128
_CROWS_PT = (_B * _H) // 128 // _NW


_FOLD = 128 // _PW


def _proj_body(t_ref, e_ref, o_ref):
    blk = t_ref.shape[0]
    t3 = t_ref[...].reshape(blk // _FOLD, _FOLD, _E)
    acc = jnp.zeros((blk // _FOLD, 128), jnp.float32)
    for a in range(_FOLD):
        acc = acc + jnp.dot(t3[:, a, :], e_ref[a],
                            preferred_element_type=jnp.float32)
    o_ref[...] = acc


def _project(table, e):
    blk = 8000
    oblk = blk // _FOLD
    return pl.pallas_call(
        _proj_body,
        grid=(_V // blk,),
        in_specs=[
            pl.BlockSpec((blk, _E), lambda i: (i, 0)),
            pl.BlockSpec((_FOLD, _E, 128), lambda i: (0, 0, 0)),
        ],
        out_specs=pl.BlockSpec((oblk, 128), lambda i: (i, 0)),
        out_shape=jax.ShapeDtypeStruct((_V // _FOLD, 128), jnp.float32),
    )(table, e)


_GPS = 8
_SB = _NG // _GPS
_SBROWS = _GPS * _CPG


def _bag_body(xr_hbm, proj_hbm, bpad_hbm, out_hbm,
              idx_v, rows_v, out_v, bias_v, gsem):
    wid = lax.axis_index("s") * _NC + lax.axis_index("c")
    pltpu.sync_copy(bpad_hbm, bias_v)
    cbase = wid * _CROWS_PT

    def superblock(sb, carry):
        pltpu.sync_copy(xr_hbm.at[pl.ds(cbase + sb * _SBROWS, _SBROWS)],
                        idx_v)

        def group(gg, carry2):
            cps = []
            for j in range(_CPG):
                cps.append(pltpu.async_copy(
                    proj_hbm.at[idx_v.at[gg * _CPG + j]],
                    rows_v.at[pl.ds(j * 128, 128)], gsem))
            for cp in cps:
                cp.wait()
            for r in range(_GR):
                def red(l8, a):
                    base = r * _H + l8 * 8
                    for u in range(8):
                        a = a + rows_v[base + u]
                    return a
                acc = lax.fori_loop(0, _H // 8, red, bias_v[...])
                out_v[sb * (_GPS * _GR) + gg * _GR + r] = acc
            return carry2

        lax.fori_loop(0, _GPS, group, 0)
        return carry

    lax.fori_loop(0, _SB, superblock, 0)
    pltpu.sync_copy(out_v, out_hbm.at[pl.ds(wid * _RPT, _RPT)])


def _bag(xr, proj, bpad):
    mesh = plsc.VectorSubcoreMesh(core_axis_name="c", subcore_axis_name="s")
    f = pl.kernel(
        _bag_body,
        mesh=mesh,
        out_type=jax.ShapeDtypeStruct((_B, _PW), jnp.float32),
        scratch_types=[
            pltpu.VMEM((_SBROWS, 128), jnp.int32),
            pltpu.VMEM((_GR * _H, _PW), jnp.float32),
            pltpu.VMEM((_RPT, _PW), jnp.float32),
            pltpu.VMEM((16,), jnp.float32),
            pltpu.SemaphoreType.DMA,
        ],
        compiler_params=pltpu.CompilerParams(use_tc_tiling_on_sc=False),
    )
    return f(xr, proj, bpad)


def kernel(x, table, W, b):
    c = W.shape[0]
    xr = x.astype(jnp.int32).reshape(-1, 128)
    wp = jnp.pad(W.T.astype(jnp.float32) * (1.0 / _H),
                 ((0, 0), (0, _PW - c)))
    bp = jnp.pad(b.astype(jnp.float32), (0, _PW - c))
    e = jnp.zeros((_FOLD, _E, 128), jnp.float32)
    for a in range(_FOLD):
        e = e.at[a, :, a * _PW:(a + 1) * _PW].set(wp)
    proj = _project(table, e).reshape(_V, _PW)
    out = _bag(xr, proj, bp)
    return out[:, :c]


# speedup vs baseline: 3.8303x; 1.1329x over previous
import jax
import jax.numpy as jnp
from jax import lax
from jax.experimental import pallas as pl
from jax.experimental.pallas import tpu as pltpu
from jax.experimental.pallas import tpu_sc as plsc

_V = 1_000_000
_E = 64
_B = 16384
_H = 200
_PW = 16

_NC = 2
_NS = 16
_NW = _NC * _NS
_RPT = _B // _NW
_GR = 16
_NG = _RPT // _GR
_CPG = (_GR * _H) // 128
_CROWS_PT = (_B * _H) // 128 // _NW


_FOLD = 128 // _PW


def _proj_body(t_ref, e_ref, o_ref):
    blk = t_ref.shape[0]
    t3 = t_ref[...].reshape(blk // _FOLD, _FOLD, _E)
    acc = jnp.zeros((blk // _FOLD, 128), jnp.float32)
    for a in range(_FOLD):
        acc = acc + jnp.dot(t3[:, a, :], e_ref[a],
                            preferred_element_type=jnp.float32)
    o_ref[...] = acc


def _project(table, e):
    blk = 8000
    oblk = blk // _FOLD
    return pl.pallas_call(
        _proj_body,
        grid=(_V // blk,),
        in_specs=[
            pl.BlockSpec((blk, _E), lambda i: (i, 0)),
            pl.BlockSpec((_FOLD, _E, 128), lambda i: (0, 0, 0)),
        ],
        out_specs=pl.BlockSpec((oblk, 128), lambda i: (i, 0)),
        out_shape=jax.ShapeDtypeStruct((_V // _FOLD, 128), jnp.float32),
    )(table, e)


_IR = 32


def _bag_body(xr_hbm, proj_hbm, bpad_hbm, out_hbm,
              idx0, idx1, rows0, rows1, out_v, bias_v,
              rsem0, rsem1, isem0, isem1):
    wid = lax.axis_index("s") * _NC + lax.axis_index("c")
    pltpu.sync_copy(bpad_hbm, bias_v)
    gbase = wid * _NG
    projv_hbm = proj_hbm
    idx = (idx0, idx1)
    rows = (rows0, rows1)
    rsem = (rsem0, rsem1)
    isem = (isem0, isem1)
    nrows = _GR * _H

    def fire_idx(g, par):
        pltpu.async_copy(xr_hbm.at[pl.ds((gbase + g) * _IR, _IR)],
                         idx[par], isem[par])

    def wait_idx(par):
        pltpu.make_async_copy(xr_hbm.at[pl.ds(0, _IR)],
                              idx[par], isem[par]).wait()

    def fire_gathers(par):
        for j in range(_CPG):
            pltpu.async_copy(projv_hbm.at[idx[par].at[j]],
                             rows[par].at[pl.ds(j * 128, 128)], rsem[par])

    def wait_gathers(par):
        pltpu.make_async_copy(projv_hbm.at[pl.ds(0, nrows)],
                              rows[par], rsem[par]).wait()

    pltpu.sync_copy(xr_hbm.at[pl.ds(gbase * _IR, _IR)], idx[0])
    fire_gathers(0)
    fire_idx(1, 1)

    zero = jnp.zeros((_PW,), jnp.float32)
    bias = bias_v[...]

    def pair(i, carry):
        for par in range(2):
            g = i * 2 + par
            nxt = g + 1

            @pl.when(nxt < _NG)
            def _():
                wait_idx(1 - par)
                fire_gathers(1 - par)

            wait_gathers(par)

            @pl.when(nxt + 1 < _NG)
            def _():
                fire_idx(nxt + 1, par)

            rv = rows[par]
            for rp in range(_GR // 2):
                b0 = 2 * rp * _H
                b1 = b0 + _H

                def red(l8, accs):
                    a = list(accs)
                    for u in range(8):
                        a[u % 4] = a[u % 4] + rv[b0 + l8 * 8 + u]
                        a[4 + u % 4] = a[4 + u % 4] + rv[b1 + l8 * 8 + u]
                    return tuple(a)

                accs = lax.fori_loop(0, _H // 8, red, (zero,) * 8)
                out_v[g * _GR + 2 * rp] = (
                    accs[0] + accs[1] + accs[2] + accs[3] + bias)
                out_v[g * _GR + 2 * rp + 1] = (
                    accs[4] + accs[5] + accs[6] + accs[7] + bias)
        return carry

    lax.fori_loop(0, _NG // 2, pair, 0)
    pltpu.sync_copy(out_v, out_hbm.at[pl.ds(wid * _RPT, _RPT)])


def _bag(xr, proj, bpad):
    mesh = plsc.VectorSubcoreMesh(core_axis_name="c", subcore_axis_name="s")
    f = pl.kernel(
        _bag_body,
        mesh=mesh,
        out_type=jax.ShapeDtypeStruct((_B, _PW), jnp.float32),
        scratch_types=[
            pltpu.VMEM((_IR, 128), jnp.int32),
            pltpu.VMEM((_IR, 128), jnp.int32),
            pltpu.VMEM((_GR * _H, _PW), jnp.float32),
            pltpu.VMEM((_GR * _H, _PW), jnp.float32),
            pltpu.VMEM((_RPT, _PW), jnp.float32),
            pltpu.VMEM((16,), jnp.float32),
            pltpu.SemaphoreType.DMA,
            pltpu.SemaphoreType.DMA,
            pltpu.SemaphoreType.DMA,
            pltpu.SemaphoreType.DMA,
        ],
        compiler_params=pltpu.CompilerParams(use_tc_tiling_on_sc=False),
    )
    return f(xr, proj, bpad)


def kernel(x, table, W, b):
    c = W.shape[0]
    ngroups = _B // _GR
    xg = x.astype(jnp.int32).reshape(ngroups, _GR * _H)
    xr = jnp.pad(xg, ((0, 0), (0, _IR * 128 - _GR * _H))).reshape(-1, 128)
    wp = jnp.pad(W.T.astype(jnp.float32) * (1.0 / _H),
                 ((0, 0), (0, _PW - c)))
    bp = jnp.pad(b.astype(jnp.float32), (0, _PW - c))
    e = jnp.zeros((_FOLD, _E, 128), jnp.float32)
    for a in range(_FOLD):
        e = e.at[a, :, a * _PW:(a + 1) * _PW].set(wp)
    proj = _project(table, e).reshape(_V, _PW)
    out = _bag(xr, proj, bp)
    return out[:, :c]


# speedup vs baseline: 4.0635x; 1.0609x over previous
import jax
import jax.numpy as jnp
from jax import lax
from jax.experimental import pallas as pl
from jax.experimental.pallas import tpu as pltpu
from jax.experimental.pallas import tpu_sc as plsc

_V = 1_000_000
_E = 64
_B = 16384
_H = 200
_PW = 16

_NC = 2
_NS = 16
_NW = _NC * _NS
_RPT = _B // _NW
_GR = 16
_NG = _RPT // _GR
_CPG = (_GR * _H) // 128
_CROWS_PT = (_B * _H) // 128 // _NW


_FOLD = 128 // _PW


def _proj_body(t_ref, w_ref, m_ref, o_ref):
    blk = t_ref.shape[0]
    r = jnp.dot(t_ref[...], w_ref[...], preferred_element_type=jnp.float32)
    r4 = r.reshape(blk // _FOLD, _FOLD, 128)
    o_ref[...] = jnp.sum(r4 * m_ref[...], axis=1)


def _project(table, wrep, mask):
    blk = 8000
    oblk = blk // _FOLD
    return pl.pallas_call(
        _proj_body,
        grid=(_V // blk,),
        in_specs=[
            pl.BlockSpec((blk, _E), lambda i: (i, 0)),
            pl.BlockSpec((_E, 128), lambda i: (0, 0)),
            pl.BlockSpec((_FOLD, 128), lambda i: (0, 0)),
        ],
        out_specs=pl.BlockSpec((oblk, 128), lambda i: (i, 0)),
        out_shape=jax.ShapeDtypeStruct((_V // _FOLD, 128), jnp.float32),
    )(table, wrep, mask)


_IR = 32


def _bag_body(xr_hbm, proj_hbm, bpad_hbm, out_hbm,
              idx0, idx1, rows0, rows1, out_v, bias_v,
              rsem0, rsem1, isem0, isem1):
    wid = lax.axis_index("s") * _NC + lax.axis_index("c")
    pltpu.sync_copy(bpad_hbm, bias_v)
    gbase = wid * _NG
    projv_hbm = proj_hbm
    idx = (idx0, idx1)
    rows = (rows0, rows1)
    rsem = (rsem0, rsem1)
    isem = (isem0, isem1)
    nrows = _GR * _H

    def fire_idx(g, par):
        pltpu.async_copy(xr_hbm.at[pl.ds((gbase + g) * _IR, _IR)],
                         idx[par], isem[par])

    def wait_idx(par):
        pltpu.make_async_copy(xr_hbm.at[pl.ds(0, _IR)],
                              idx[par], isem[par]).wait()

    def fire_gathers(par):
        for j in range(_CPG):
            pltpu.async_copy(projv_hbm.at[idx[par].at[j]],
                             rows[par].at[pl.ds(j * 128, 128)], rsem[par])

    def wait_gathers(par):
        pltpu.make_async_copy(projv_hbm.at[pl.ds(0, nrows)],
                              rows[par], rsem[par]).wait()

    pltpu.sync_copy(xr_hbm.at[pl.ds(gbase * _IR, _IR)], idx[0])
    fire_gathers(0)
    fire_idx(1, 1)

    zero = jnp.zeros((_PW,), jnp.float32)
    bias = bias_v[...]

    def pair(i, carry):
        for par in range(2):
            g = i * 2 + par
            nxt = g + 1

            @pl.when(nxt < _NG)
            def _():
                wait_idx(1 - par)
                fire_gathers(1 - par)

            wait_gathers(par)

            @pl.when(nxt + 1 < _NG)
            def _():
                fire_idx(nxt + 1, par)

            rv = rows[par]
            for rp in range(_GR // 2):
                b0 = 2 * rp * _H
                b1 = b0 + _H

                def red(l8, accs):
                    a = list(accs)
                    for u in range(8):
                        a[u % 4] = a[u % 4] + rv[b0 + l8 * 8 + u]
                        a[4 + u % 4] = a[4 + u % 4] + rv[b1 + l8 * 8 + u]
                    return tuple(a)

                accs = lax.fori_loop(0, _H // 8, red, (zero,) * 8)
                out_v[g * _GR + 2 * rp] = (
                    accs[0] + accs[1] + accs[2] + accs[3] + bias)
                out_v[g * _GR + 2 * rp + 1] = (
                    accs[4] + accs[5] + accs[6] + accs[7] + bias)
        return carry

    lax.fori_loop(0, _NG // 2, pair, 0)
    pltpu.sync_copy(out_v, out_hbm.at[pl.ds(wid * _RPT, _RPT)])


def _bag(xr, proj, bpad):
    mesh = plsc.VectorSubcoreMesh(core_axis_name="c", subcore_axis_name="s")
    f = pl.kernel(
        _bag_body,
        mesh=mesh,
        out_type=jax.ShapeDtypeStruct((_B, _PW), jnp.float32),
        scratch_types=[
            pltpu.VMEM((_IR, 128), jnp.int32),
            pltpu.VMEM((_IR, 128), jnp.int32),
            pltpu.VMEM((_GR * _H, _PW), jnp.float32),
            pltpu.VMEM((_GR * _H, _PW), jnp.float32),
            pltpu.VMEM((_RPT, _PW), jnp.float32),
            pltpu.VMEM((16,), jnp.float32),
            pltpu.SemaphoreType.DMA,
            pltpu.SemaphoreType.DMA,
            pltpu.SemaphoreType.DMA,
            pltpu.SemaphoreType.DMA,
        ],
        compiler_params=pltpu.CompilerParams(use_tc_tiling_on_sc=False),
    )
    return f(xr, proj, bpad)


def kernel(x, table, W, b):
    c = W.shape[0]
    ngroups = _B // _GR
    xg = x.astype(jnp.int32).reshape(ngroups, _GR * _H)
    xr = jnp.pad(xg, ((0, 0), (0, _IR * 128 - _GR * _H))).reshape(-1, 128)
    wp = jnp.pad(W.T.astype(jnp.float32) * (1.0 / _H),
                 ((0, 0), (0, _PW - c)))
    bp = jnp.pad(b.astype(jnp.float32), (0, _PW - c))
    wrep = jnp.tile(wp, (1, _FOLD))
    lane = jnp.arange(128) // _PW
    mask = (lane[None, :] == jnp.arange(_FOLD)[:, None]).astype(jnp.float32)
    proj = _project(table, wrep, mask).reshape(_V, _PW)
    out = _bag(xr, proj, bp)
    return out[:, :c]


# speedup vs baseline: 8.5389x; 2.1014x over previous
import jax
import jax.numpy as jnp
from jax import lax
from jax.experimental import pallas as pl
from jax.experimental.pallas import tpu as pltpu
from jax.experimental.pallas import tpu_sc as plsc

_V = 1_000_000
_E = 64
_B = 16384
_H = 200
_PW = 16

_NC = 2
_NS = 16
_NW = _NC * _NS
_RPT = _B // _NW
_GR = 16
_NG = _RPT // _GR
_CPG = (_GR * _H) // 128
_CROWS_PT = (_B * _H) // 128 // _NW


_FOLD = 128 // _PW


def _proj_body(t_ref, w_ref, m_ref, o_ref):
    blk = t_ref.shape[1]
    r = jax.lax.dot_general(t_ref[...], w_ref[...],
                            (((0,), (0,)), ((), ())),
                            preferred_element_type=jnp.float32)
    r4 = r.reshape(blk // _FOLD, _FOLD, 128)
    o_ref[...] = jnp.sum(r4 * m_ref[...], axis=1)


def _project(table_t, wrep, mask):
    blk = 16384
    oblk = blk // _FOLD
    return pl.pallas_call(
        _proj_body,
        grid=(-(-_V // blk),),
        in_specs=[
            pl.BlockSpec((_E, blk), lambda i: (0, i)),
            pl.BlockSpec((_E, 128), lambda i: (0, 0)),
            pl.BlockSpec((_FOLD, 128), lambda i: (0, 0)),
        ],
        out_specs=pl.BlockSpec((oblk, 128), lambda i: (i, 0)),
        out_shape=jax.ShapeDtypeStruct((_V // _FOLD, 128), jnp.float32),
    )(table_t, wrep, mask)


_IR = 32


def _bag_body(xr_hbm, proj_hbm, bpad_hbm, out_hbm,
              idx0, idx1, rows0, rows1, out_v, bias_v,
              rsem0, rsem1, isem0, isem1):
    wid = lax.axis_index("s") * _NC + lax.axis_index("c")
    pltpu.sync_copy(bpad_hbm, bias_v)
    gbase = wid * _NG
    projv_hbm = proj_hbm
    idx = (idx0, idx1)
    rows = (rows0, rows1)
    rsem = (rsem0, rsem1)
    isem = (isem0, isem1)
    nrows = _GR * _H

    def fire_idx(g, par):
        pltpu.async_copy(xr_hbm.at[pl.ds((gbase + g) * _IR, _IR)],
                         idx[par], isem[par])

    def wait_idx(par):
        pltpu.make_async_copy(xr_hbm.at[pl.ds(0, _IR)],
                              idx[par], isem[par]).wait()

    def fire_gathers(par):
        for j in range(_CPG):
            pltpu.async_copy(projv_hbm.at[idx[par].at[j]],
                             rows[par].at[pl.ds(j * 128, 128)], rsem[par])

    def wait_gathers(par):
        pltpu.make_async_copy(projv_hbm.at[pl.ds(0, nrows)],
                              rows[par], rsem[par]).wait()

    pltpu.sync_copy(xr_hbm.at[pl.ds(gbase * _IR, _IR)], idx[0])
    fire_gathers(0)
    fire_idx(1, 1)

    zero = jnp.zeros((_PW,), jnp.float32)
    bias = bias_v[...]

    def pair(i, carry):
        for par in range(2):
            g = i * 2 + par
            nxt = g + 1

            @pl.when(nxt < _NG)
            def _():
                wait_idx(1 - par)
                fire_gathers(1 - par)

            wait_gathers(par)

            @pl.when(nxt + 1 < _NG)
            def _():
                fire_idx(nxt + 1, par)

            rv = rows[par]
            for rp in range(_GR // 2):
                b0 = 2 * rp * _H
                b1 = b0 + _H

                def red(l8, accs):
                    a = list(accs)
                    for u in range(8):
                        a[u % 4] = a[u % 4] + rv[b0 + l8 * 8 + u]
                        a[4 + u % 4] = a[4 + u % 4] + rv[b1 + l8 * 8 + u]
                    return tuple(a)

                accs = lax.fori_loop(0, _H // 8, red, (zero,) * 8)
                out_v[g * _GR + 2 * rp] = (
                    accs[0] + accs[1] + accs[2] + accs[3] + bias)
                out_v[g * _GR + 2 * rp + 1] = (
                    accs[4] + accs[5] + accs[6] + accs[7] + bias)
        return carry

    lax.fori_loop(0, _NG // 2, pair, 0)
    pltpu.sync_copy(out_v, out_hbm.at[pl.ds(wid * _RPT, _RPT)])


def _bag(xr, proj, bpad):
    mesh = plsc.VectorSubcoreMesh(core_axis_name="c", subcore_axis_name="s")
    f = pl.kernel(
        _bag_body,
        mesh=mesh,
        out_type=jax.ShapeDtypeStruct((_B, _PW), jnp.float32),
        scratch_types=[
            pltpu.VMEM((_IR, 128), jnp.int32),
            pltpu.VMEM((_IR, 128), jnp.int32),
            pltpu.VMEM((_GR * _H, _PW), jnp.float32),
            pltpu.VMEM((_GR * _H, _PW), jnp.float32),
            pltpu.VMEM((_RPT, _PW), jnp.float32),
            pltpu.VMEM((16,), jnp.float32),
            pltpu.SemaphoreType.DMA,
            pltpu.SemaphoreType.DMA,
            pltpu.SemaphoreType.DMA,
            pltpu.SemaphoreType.DMA,
        ],
        compiler_params=pltpu.CompilerParams(use_tc_tiling_on_sc=False),
    )
    return f(xr, proj, bpad)


def kernel(x, table, W, b):
    c = W.shape[0]
    ngroups = _B // _GR
    xg = x.astype(jnp.int32).reshape(ngroups, _GR * _H)
    xr = jnp.pad(xg, ((0, 0), (0, _IR * 128 - _GR * _H))).reshape(-1, 128)
    wp = jnp.pad(W.T.astype(jnp.float32) * (1.0 / _H),
                 ((0, 0), (0, _PW - c)))
    bp = jnp.pad(b.astype(jnp.float32), (0, _PW - c))
    wrep = jnp.tile(wp, (1, _FOLD))
    lane = jnp.arange(128) // _PW
    mask = (lane[None, :] == jnp.arange(_FOLD)[:, None]).astype(jnp.float32)
    proj = _project(table.T, wrep, mask).reshape(_V, _PW)
    out = _bag(xr, proj, bp)
    return out[:, :c]


# speedup vs baseline: 8.8651x; 1.0382x over previous
import jax
import jax.numpy as jnp
from jax import lax
from jax.experimental import pallas as pl
from jax.experimental.pallas import tpu as pltpu
from jax.experimental.pallas import tpu_sc as plsc

_V = 1_000_000
_E = 64
_B = 16384
_H = 200
_PW = 16

_NC = 2
_NS = 16
_NW = _NC * _NS
_RPT = _B // _NW
_GR = 16
_NG = _RPT // _GR
_CPG = (_GR * _H) // 128
_CROWS_PT = (_B * _H) // 128 // _NW


_FOLD = 128 // _PW


def _proj_body(t_ref, w_ref, m_ref, o_ref):
    blk = t_ref.shape[1]
    r = jax.lax.dot_general(t_ref[...], w_ref[...],
                            (((0,), (0,)), ((), ())),
                            preferred_element_type=jnp.float32)
    r4 = r.reshape(blk // _FOLD, _FOLD, 128)
    o_ref[...] = jnp.sum(r4 * m_ref[...], axis=1)


def _project(table_t, wrep, mask):
    blk = 32768
    oblk = blk // _FOLD
    return pl.pallas_call(
        _proj_body,
        grid=(-(-_V // blk),),
        in_specs=[
            pl.BlockSpec((_E, blk), lambda i: (0, i)),
            pl.BlockSpec((_E, 128), lambda i: (0, 0)),
            pl.BlockSpec((_FOLD, 128), lambda i: (0, 0)),
        ],
        out_specs=pl.BlockSpec((oblk, 128), lambda i: (i, 0)),
        out_shape=jax.ShapeDtypeStruct((_V // _FOLD, 128), jnp.float32),
    )(table_t, wrep, mask)


_IR = 32


def _bag_body(xr_hbm, proj_hbm, bpad_hbm, out_hbm,
              idx0, idx1, rows0, rows1, out_v, bias_v,
              rsem0, rsem1, isem0, isem1):
    wid = lax.axis_index("s") * _NC + lax.axis_index("c")
    pltpu.sync_copy(bpad_hbm, bias_v)
    gbase = wid * _NG
    projv_hbm = proj_hbm
    idx = (idx0, idx1)
    rows = (rows0, rows1)
    rsem = (rsem0, rsem1)
    isem = (isem0, isem1)
    nrows = _GR * _H

    def fire_idx(g, par):
        pltpu.async_copy(xr_hbm.at[pl.ds((gbase + g) * _IR, _IR)],
                         idx[par], isem[par])

    def wait_idx(par):
        pltpu.make_async_copy(xr_hbm.at[pl.ds(0, _IR)],
                              idx[par], isem[par]).wait()

    def fire_gathers(par):
        for j in range(_CPG):
            pltpu.async_copy(projv_hbm.at[idx[par].at[j]],
                             rows[par].at[pl.ds(j * 128, 128)], rsem[par])

    def wait_gathers(par):
        pltpu.make_async_copy(projv_hbm.at[pl.ds(0, nrows)],
                              rows[par], rsem[par]).wait()

    pltpu.sync_copy(xr_hbm.at[pl.ds(gbase * _IR, _IR)], idx[0])
    fire_gathers(0)
    fire_idx(1, 1)

    zero = jnp.zeros((_PW,), jnp.float32)
    bias = bias_v[...]

    def pair(i, carry):
        for par in range(2):
            g = i * 2 + par
            nxt = g + 1

            @pl.when(nxt < _NG)
            def _():
                wait_idx(1 - par)
                fire_gathers(1 - par)

            wait_gathers(par)

            @pl.when(nxt + 1 < _NG)
            def _():
                fire_idx(nxt + 1, par)

            rv = rows[par]
            for rp in range(_GR // 2):
                b0 = 2 * rp * _H
                b1 = b0 + _H

                def red(l8, accs):
                    a = list(accs)
                    for u in range(8):
                        a[u % 4] = a[u % 4] + rv[b0 + l8 * 8 + u]
                        a[4 + u % 4] = a[4 + u % 4] + rv[b1 + l8 * 8 + u]
                    return tuple(a)

                accs = lax.fori_loop(0, _H // 8, red, (zero,) * 8)
                out_v[g * _GR + 2 * rp] = (
                    accs[0] + accs[1] + accs[2] + accs[3] + bias)
                out_v[g * _GR + 2 * rp + 1] = (
                    accs[4] + accs[5] + accs[6] + accs[7] + bias)
        return carry

    lax.fori_loop(0, _NG // 2, pair, 0)
    pltpu.sync_copy(out_v, out_hbm.at[pl.ds(wid * _RPT, _RPT)])


def _bag(xr, proj, bpad):
    mesh = plsc.VectorSubcoreMesh(core_axis_name="c", subcore_axis_name="s")
    f = pl.kernel(
        _bag_body,
        mesh=mesh,
        out_type=jax.ShapeDtypeStruct((_B, _PW), jnp.float32),
        scratch_types=[
            pltpu.VMEM((_IR, 128), jnp.int32),
            pltpu.VMEM((_IR, 128), jnp.int32),
            pltpu.VMEM((_GR * _H, _PW), jnp.float32),
            pltpu.VMEM((_GR * _H, _PW), jnp.float32),
            pltpu.VMEM((_RPT, _PW), jnp.float32),
            pltpu.VMEM((16,), jnp.float32),
            pltpu.SemaphoreType.DMA,
            pltpu.SemaphoreType.DMA,
            pltpu.SemaphoreType.DMA,
            pltpu.SemaphoreType.DMA,
        ],
        compiler_params=pltpu.CompilerParams(use_tc_tiling_on_sc=False),
    )
    return f(xr, proj, bpad)


def kernel(x, table, W, b):
    c = W.shape[0]
    ngroups = _B // _GR
    xg = x.astype(jnp.int32).reshape(ngroups, _GR * _H)
    xr = jnp.pad(xg, ((0, 0), (0, _IR * 128 - _GR * _H))).reshape(-1, 128)
    wp = jnp.pad(W.T.astype(jnp.float32) * (1.0 / _H),
                 ((0, 0), (0, _PW - c)))
    bp = jnp.pad(b.astype(jnp.float32), (0, _PW - c))
    wrep = jnp.tile(wp, (1, _FOLD))
    lane = jnp.arange(128) // _PW
    mask = (lane[None, :] == jnp.arange(_FOLD)[:, None]).astype(jnp.float32)
    proj = _project(table.T, wrep, mask).reshape(_V, _PW)
    out = _bag(xr, proj, bp)
    return out[:, :c]


# speedup vs baseline: 10.2152x; 1.1523x over previous
import jax
import jax.numpy as jnp
from jax import lax
from jax.experimental import pallas as pl
from jax.experimental.pallas import tpu as pltpu
from jax.experimental.pallas import tpu_sc as plsc

_V = 1_000_000
_E = 64
_B = 16384
_H = 200
_PW = 16

_NC = 2
_NS = 16
_NW = _NC * _NS
_RPT = _B // _NW
_GR = 16
_NG = _RPT // _GR
_CPG = (_GR * _H) // 128
_CROWS_PT = (_B * _H) // 128 // _NW


_FOLD = 128 // _PW


def _proj_body(t_ref, w_ref, m_ref, o_ref):
    blk = t_ref.shape[1]
    r = jax.lax.dot_general(t_ref[...], w_ref[...],
                            (((0,), (0,)), ((), ())),
                            preferred_element_type=jnp.float32)
    r4 = r.reshape(blk // _FOLD, _FOLD, 128)
    o_ref[...] = jnp.sum(r4 * m_ref[...], axis=1)


def _project(table_t, wrep, mask):
    blk = 32768
    oblk = blk // _FOLD
    return pl.pallas_call(
        _proj_body,
        grid=(-(-_V // blk),),
        in_specs=[
            pl.BlockSpec((_E, blk), lambda i: (0, i)),
            pl.BlockSpec((_E, 128), lambda i: (0, 0)),
            pl.BlockSpec((_FOLD, 128), lambda i: (0, 0)),
        ],
        out_specs=pl.BlockSpec((oblk, 128), lambda i: (i, 0)),
        out_shape=jax.ShapeDtypeStruct((_V // _FOLD, 128), jnp.float32),
    )(table_t, wrep, mask)


_IR = 32


def _transpose_idx(idx2, idxlin):
    offs = lax.iota(jnp.int32, 16) * _H

    def step(l, carry):
        plsc.store_scatter(idxlin, [offs + l], idx2[l])
        return carry

    lax.fori_loop(0, _H, step, 0)


def _bag_body(xt_hbm, proj_hbm, bpad_hbm, out_hbm,
              idx0, idx1, lin0, lin1, rows0, rows1, out_v, bias_v,
              rsem0, rsem1, isem0, isem1):
    wid = lax.axis_index("s") * _NC + lax.axis_index("c")
    pltpu.sync_copy(bpad_hbm, bias_v)
    gbase = wid * _NG
    projv_hbm = proj_hbm
    idx = (idx0, idx1)
    lin = (lin0, lin1)
    rows = (rows0, rows1)
    rsem = (rsem0, rsem1)
    isem = (isem0, isem1)
    nrows = _GR * _H

    def fire_idx(g, par):
        pltpu.async_copy(
            xt_hbm.at[pl.ds(0, _H), pl.ds((gbase + g) * _GR, _GR)],
            idx[par], isem[par])

    def wait_idx(par):
        pltpu.make_async_copy(xt_hbm.at[pl.ds(0, _H), pl.ds(0, _GR)],
                              idx[par], isem[par]).wait()

    def fire_gathers(par):
        for j in range(_CPG):
            pltpu.async_copy(projv_hbm.at[lin[par].at[pl.ds(j * 128, 128)]],
                             rows[par].at[pl.ds(j * 128, 128)], rsem[par])

    def wait_gathers(par):
        pltpu.make_async_copy(projv_hbm.at[pl.ds(0, nrows)],
                              rows[par], rsem[par]).wait()

    pltpu.sync_copy(xt_hbm.at[pl.ds(0, _H), pl.ds(gbase * _GR, _GR)], idx[0])
    _transpose_idx(idx[0], lin[0])
    fire_gathers(0)
    fire_idx(1, 1)

    zero = jnp.zeros((_PW,), jnp.float32)
    bias = bias_v[...]

    def pair(i, carry):
        for par in range(2):
            g = i * 2 + par
            nxt = g + 1

            @pl.when(nxt < _NG)
            def _():
                wait_idx(1 - par)
                _transpose_idx(idx[1 - par], lin[1 - par])
                fire_gathers(1 - par)

            wait_gathers(par)

            @pl.when(nxt + 1 < _NG)
            def _():
                fire_idx(nxt + 1, par)

            rv = rows[par]
            for rp in range(_GR // 2):
                b0 = 2 * rp * _H
                b1 = b0 + _H

                def red(l8, accs):
                    a = list(accs)
                    for u in range(8):
                        a[u % 4] = a[u % 4] + rv[b0 + l8 * 8 + u]
                        a[4 + u % 4] = a[4 + u % 4] + rv[b1 + l8 * 8 + u]
                    return tuple(a)

                accs = lax.fori_loop(0, _H // 8, red, (zero,) * 8)
                out_v[g * _GR + 2 * rp] = (
                    accs[0] + accs[1] + accs[2] + accs[3] + bias)
                out_v[g * _GR + 2 * rp + 1] = (
                    accs[4] + accs[5] + accs[6] + accs[7] + bias)
        return carry

    lax.fori_loop(0, _NG // 2, pair, 0)
    pltpu.sync_copy(out_v, out_hbm.at[pl.ds(wid * _RPT, _RPT)])


def _bag(xt, proj, bpad):
    mesh = plsc.VectorSubcoreMesh(core_axis_name="c", subcore_axis_name="s")
    f = pl.kernel(
        _bag_body,
        mesh=mesh,
        out_type=jax.ShapeDtypeStruct((_B, _PW), jnp.float32),
        scratch_types=[
            pltpu.VMEM((_H, _GR), jnp.int32),
            pltpu.VMEM((_H, _GR), jnp.int32),
            pltpu.VMEM((_GR * _H,), jnp.int32),
            pltpu.VMEM((_GR * _H,), jnp.int32),
            pltpu.VMEM((_GR * _H, _PW), jnp.float32),
            pltpu.VMEM((_GR * _H, _PW), jnp.float32),
            pltpu.VMEM((_RPT, _PW), jnp.float32),
            pltpu.VMEM((16,), jnp.float32),
            pltpu.SemaphoreType.DMA,
            pltpu.SemaphoreType.DMA,
            pltpu.SemaphoreType.DMA,
            pltpu.SemaphoreType.DMA,
        ],
        compiler_params=pltpu.CompilerParams(use_tc_tiling_on_sc=False,
                                             needs_layout_passes=False),
    )
    return f(xt, proj, bpad)


def kernel(x, table, W, b):
    c = W.shape[0]
    xt = x.astype(jnp.int32).T
    wp = jnp.pad(W.T.astype(jnp.float32) * (1.0 / _H),
                 ((0, 0), (0, _PW - c)))
    bp = jnp.pad(b.astype(jnp.float32), (0, _PW - c))
    wrep = jnp.tile(wp, (1, _FOLD))
    lane = jnp.arange(128) // _PW
    mask = (lane[None, :] == jnp.arange(_FOLD)[:, None]).astype(jnp.float32)
    proj = _project(table.T, wrep, mask).reshape(_V, _PW)
    out = _bag(xt, proj, bp)
    return out[:, :c]


# speedup vs baseline: 10.6493x; 1.0425x over previous
import jax
import jax.numpy as jnp
from jax import lax
from jax.experimental import pallas as pl
from jax.experimental.pallas import tpu as pltpu
from jax.experimental.pallas import tpu_sc as plsc

_V = 1_000_000
_E = 64
_B = 16384
_H = 200
_PW = 16

_NC = 2
_NS = 16
_NW = _NC * _NS
_RPT = _B // _NW
_GR = 16
_NG = _RPT // _GR
_CPG = (_GR * _H) // 128
_CROWS_PT = (_B * _H) // 128 // _NW


_FOLD = 128 // _PW


def _proj_body(t_ref, w_ref, m_ref, o_ref):
    blk = t_ref.shape[1]
    r = jax.lax.dot_general(t_ref[...], w_ref[...],
                            (((0,), (0,)), ((), ())),
                            preferred_element_type=jnp.float32)
    r4 = r.reshape(blk // _FOLD, _FOLD, 128)
    o_ref[...] = jnp.sum(r4 * m_ref[...], axis=1)


def _project(table_t, wrep, mask):
    blk = 32768
    oblk = blk // _FOLD
    return pl.pallas_call(
        _proj_body,
        grid=(-(-_V // blk),),
        in_specs=[
            pl.BlockSpec((_E, blk), lambda i: (0, i)),
            pl.BlockSpec((_E, 128), lambda i: (0, 0)),
            pl.BlockSpec((_FOLD, 128), lambda i: (0, 0)),
        ],
        out_specs=pl.BlockSpec((oblk, 128), lambda i: (i, 0)),
        out_shape=jax.ShapeDtypeStruct((_V // _FOLD, 128), jnp.float32),
    )(table_t, wrep, mask)


_DO = 2


def _transpose_idx(idx2, idxlin):
    offs = lax.iota(jnp.int32, 16) * _H

    def step(l, carry):
        plsc.store_scatter(idxlin, [offs + l], idx2[l])
        return carry

    lax.fori_loop(0, _H, step, 0)


def _bag_body(xt_hbm, proj_hbm, bpad_hbm, out_hbm,
              idx0, idx1, lin0, lin1, rows0, rows1, out_v, bias_v,
              rsem0, rsem1, isem0, isem1):
    wid = lax.axis_index("s") * _NC + lax.axis_index("c")
    pltpu.sync_copy(bpad_hbm, bias_v)
    gbase = wid * _NG
    projv_hbm = proj_hbm
    idx = (idx0, idx1)
    lin = (lin0, lin1)
    rows = (rows0, rows1)
    rsem = (rsem0, rsem1)
    isem = (isem0, isem1)
    nrows = _GR * _H

    def fire_idx(g, par):
        pltpu.async_copy(
            xt_hbm.at[pl.ds(0, _H), pl.ds((gbase + g) * _GR, _GR)],
            idx[par], isem[par])

    def wait_idx(par):
        pltpu.make_async_copy(xt_hbm.at[pl.ds(0, _H), pl.ds(0, _GR)],
                              idx[par], isem[par]).wait()

    def fire_gathers(par):
        for j in range(_CPG):
            pltpu.async_copy(projv_hbm.at[lin[par].at[pl.ds(j * 128, 128)]],
                             rows[par].at[pl.ds(j * 128, 128)], rsem[par])

    def wait_gathers(par):
        pltpu.make_async_copy(projv_hbm.at[pl.ds(0, nrows)],
                              rows[par], rsem[par]).wait()

    pltpu.sync_copy(xt_hbm.at[pl.ds(0, _H), pl.ds(gbase * _GR, _GR)], idx[0])
    _transpose_idx(idx[0], lin[0])
    fire_gathers(0)
    fire_idx(1, 1)

    zero = jnp.zeros((_PW,), jnp.float32)
    bias = bias_v[...]
    lanes = lax.iota(jnp.int32, 16)
    omask = lanes < _DO

    def pair(i, carry):
        for par in range(2):
            g = i * 2 + par
            nxt = g + 1

            @pl.when(nxt < _NG)
            def _():
                wait_idx(1 - par)
                _transpose_idx(idx[1 - par], lin[1 - par])
                fire_gathers(1 - par)

            wait_gathers(par)

            @pl.when(nxt + 1 < _NG)
            def _():
                fire_idx(nxt + 1, par)

            rv = rows[par]
            for rp in range(_GR // 2):
                b0 = 2 * rp * _H
                b1 = b0 + _H

                def red(l8, accs):
                    a = list(accs)
                    for u in range(8):
                        a[u % 4] = a[u % 4] + rv[b0 + l8 * 8 + u]
                        a[4 + u % 4] = a[4 + u % 4] + rv[b1 + l8 * 8 + u]
                    return tuple(a)

                accs = lax.fori_loop(0, _H // 8, red, (zero,) * 8)
                a0 = accs[0] + accs[1] + accs[2] + accs[3] + bias
                a1 = accs[4] + accs[5] + accs[6] + accs[7] + bias
                rloc = g * _GR + 2 * rp
                plsc.store_scatter(out_v, [lanes, jnp.full((16,), 0, jnp.int32) + rloc],
                                   a0, mask=omask)
                plsc.store_scatter(out_v, [lanes, jnp.full((16,), 1, jnp.int32) + rloc],
                                   a1, mask=omask)
        return carry

    lax.fori_loop(0, _NG // 2, pair, 0)
    pltpu.sync_copy(out_v,
                    out_hbm.at[pl.ds(0, _DO), pl.ds(wid * _RPT, _RPT)])


def _bag(xt, proj, bpad):
    mesh = plsc.VectorSubcoreMesh(core_axis_name="c", subcore_axis_name="s")
    f = pl.kernel(
        _bag_body,
        mesh=mesh,
        out_type=jax.ShapeDtypeStruct((_DO, _B), jnp.float32),
        scratch_types=[
            pltpu.VMEM((_H, _GR), jnp.int32),
            pltpu.VMEM((_H, _GR), jnp.int32),
            pltpu.VMEM((_GR * _H,), jnp.int32),
            pltpu.VMEM((_GR * _H,), jnp.int32),
            pltpu.VMEM((_GR * _H, _PW), jnp.float32),
            pltpu.VMEM((_GR * _H, _PW), jnp.float32),
            pltpu.VMEM((_DO, _RPT), jnp.float32),
            pltpu.VMEM((16,), jnp.float32),
            pltpu.SemaphoreType.DMA,
            pltpu.SemaphoreType.DMA,
            pltpu.SemaphoreType.DMA,
            pltpu.SemaphoreType.DMA,
        ],
        compiler_params=pltpu.CompilerParams(use_tc_tiling_on_sc=False,
                                             needs_layout_passes=False),
    )
    return f(xt, proj, bpad)


def kernel(x, table, W, b):
    c = W.shape[0]
    xt = x.astype(jnp.int32).T
    wp = jnp.pad(W.T.astype(jnp.float32) * (1.0 / _H),
                 ((0, 0), (0, _PW - c)))
    bp = jnp.pad(b.astype(jnp.float32), (0, _PW - c))
    wrep = jnp.tile(wp, (1, _FOLD))
    lane = jnp.arange(128) // _PW
    mask = (lane[None, :] == jnp.arange(_FOLD)[:, None]).astype(jnp.float32)
    proj = _project(table.T, wrep, mask).reshape(_V, _PW)
    out = _bag(xt, proj, bp)
    return out.T
